# Initial kernel scaffold; baseline (speedup 1.0000x reference)
import jax
import jax.numpy as jnp
from jax.experimental import pallas as pl


def kernel(h, edge_index, norm, W, b):
    raise NotImplementedError("write your pallas kernel here")


# speedup vs baseline: 9.3436x; 9.3436x over previous
import functools

import jax
import jax.numpy as jnp
from jax import lax
from jax.experimental import pallas as pl
from jax.experimental.pallas import tpu as pltpu
from jax.experimental.pallas import tpu_sc as plsc

N_NODES = 10000
F = 256
HALF = 128
NSUB = 16
NCORE = 2
CH = 128
NPASS = 2
NPP = N_NODES // NPASS
ROWS_PER_TILE = 320
N_ACC = NSUB * ROWS_PER_TILE
N_DUMMY = N_ACC - NPP


def _mm_body(h_ref, w_ref, n_ref, o_ref):
    hw = jnp.dot(h_ref[...], w_ref[...], preferred_element_type=jnp.float32)
    hw = hw * n_ref[...]
    o_ref[0, :, :] = hw[:, :HALF]
    o_ref[1, :, :] = hw[:, HALF:]


def _transform(h, W, norm):
    R = 1000
    return pl.pallas_call(
        _mm_body,
        grid=(N_NODES // R,),
        in_specs=[
            pl.BlockSpec((R, F), lambda i: (i, 0)),
            pl.BlockSpec((F, F), lambda i: (0, 0)),
            pl.BlockSpec((R, 1), lambda i: (i, 0)),
        ],
        out_specs=pl.BlockSpec((NCORE, R, HALF), lambda i: (0, i, 0)),
        out_shape=jax.ShapeDtypeStruct((NCORE, N_NODES, HALF), jnp.float32),
    )(h, W, norm)


def _aggregate(hw2, src_p, dst_p, zrows, n_chunks):
    mesh = plsc.VectorSubcoreMesh(core_axis_name="c", subcore_axis_name="s")

    @functools.partial(
        pl.kernel,
        out_type=jax.ShapeDtypeStruct((NCORE, N_NODES, HALF), jnp.float32),
        mesh=mesh,
        scratch_types=[
            pltpu.VMEM((n_chunks, CH), jnp.int32),
            pltpu.VMEM((n_chunks, CH), jnp.int32),
            pltpu.VMEM((CH, HALF), jnp.float32),
            pltpu.VMEM((CH, HALF), jnp.float32),
            pltpu.VMEM_SHARED((N_ACC, HALF), jnp.float32),
            pltpu.SemaphoreType.DMA,
            pltpu.SemaphoreType.DMA,
        ],
    )
    def agg(hw_hbm, src_hbm, dst_hbm, z_hbm, out_hbm,
            src_v, dst_v, buf0, buf1, acc, g0, g1):
        cid = lax.axis_index("c")
        sid = lax.axis_index("s")
        base = sid * ROWS_PER_TILE

        pltpu.sync_copy(src_hbm.at[sid], src_v)

        hw_view = hw_hbm.at[cid]
        bufs = (buf0, buf1)
        sems = (g0, g1)

        for p in range(NPASS):
            pltpu.sync_copy(z_hbm, acc.at[pl.ds(base, ROWS_PER_TILE)])
            pltpu.sync_copy(dst_hbm.at[p].at[sid], dst_v)
            plsc.subcore_barrier()

            pltpu.async_copy(hw_view.at[src_v.at[0]], buf0, g0)
            pltpu.async_copy(hw_view.at[src_v.at[1]], buf1, g1)

            @pl.loop(0, n_chunks, step=2)
            def _(j):
                for bsel in range(2):
                    jj = j + bsel
                    buf, sem = bufs[bsel], sems[bsel]
                    pltpu.make_async_copy(
                        hw_view.at[src_v.at[jj]], buf, sem).wait()
                    pltpu.sync_copy(buf, acc.at[dst_v.at[jj]], add=True)

                    @pl.when(jj + 2 < n_chunks)
                    def _():
                        pltpu.async_copy(hw_view.at[src_v.at[jj + 2]], buf, sem)

            plsc.subcore_barrier()
            out_view = out_hbm.at[cid]

            @pl.when(sid < NSUB - 1)
            def _():
                pltpu.sync_copy(
                    acc.at[pl.ds(base, ROWS_PER_TILE)],
                    out_view.at[pl.ds(p * NPP + base, ROWS_PER_TILE)])

            @pl.when(sid == NSUB - 1)
            def _():
                last = NPP - (NSUB - 1) * ROWS_PER_TILE
                pltpu.sync_copy(
                    acc.at[pl.ds(base, last)],
                    out_view.at[pl.ds(p * NPP + base, last)])

            plsc.subcore_barrier()

    return agg(hw2, src_p, dst_p, zrows)


def _finish_body(a_ref, n_ref, b_ref, o_ref):
    n = n_ref[...]
    bvec = b_ref[...]
    o_ref[:, :HALF] = jnp.maximum(a_ref[0, :, :] * n + bvec[:, :HALF], 0.0)
    o_ref[:, HALF:] = jnp.maximum(a_ref[1, :, :] * n + bvec[:, HALF:], 0.0)


def _finish(agg2, norm, b):
    R = 1000
    b2 = b.reshape(1, F)
    return pl.pallas_call(
        _finish_body,
        grid=(N_NODES // R,),
        in_specs=[
            pl.BlockSpec((NCORE, R, HALF), lambda i: (0, i, 0)),
            pl.BlockSpec((R, 1), lambda i: (i, 0)),
            pl.BlockSpec((1, F), lambda i: (0, 0)),
        ],
        out_specs=pl.BlockSpec((R, F), lambda i: (i, 0)),
        out_shape=jax.ShapeDtypeStruct((N_NODES, F), jnp.float32),
    )(agg2, norm, b2)


def kernel(h, edge_index, norm, W, b):
    src = edge_index[0].astype(jnp.int32)
    dst = edge_index[1].astype(jnp.int32)
    E = src.shape[0]

    per_tile = NSUB * CH
    n_chunks = -(-E // per_tile)
    n_chunks += n_chunks % 2
    e_pad = n_chunks * per_tile
    pad = e_pad - E
    ar = jnp.arange(pad, dtype=jnp.int32)
    src_p = jnp.concatenate([src, ar % NSUB]).reshape(NSUB, n_chunks, CH)

    ar_all = jnp.arange(e_pad, dtype=jnp.int32)
    dst_full = jnp.concatenate([dst, jnp.full((pad,), -1, jnp.int32)])
    dst_passes = []
    for p in range(NPASS):
        local = dst_full - p * NPP
        inrange = (local >= 0) & (local < NPP)
        dst_passes.append(
            jnp.where(inrange, local, NPP + (ar_all % N_DUMMY)))
    dst_p = jnp.stack(dst_passes).reshape(NPASS, NSUB, n_chunks, CH)

    zrows = jnp.zeros((ROWS_PER_TILE, HALF), jnp.float32)

    hw2 = _transform(h, W, norm)
    agg2 = _aggregate(hw2, src_p, dst_p, zrows, n_chunks)
    return _finish(agg2, norm, b)
